# baseline (speedup 1.0000x reference)
import functools

import jax
import jax.numpy as jnp
from jax import lax
from jax.experimental import pallas as pl
from jax.experimental.pallas import tpu as pltpu
from jax.experimental.pallas import tpu_sc as plsc

MAXSPAN = 255
QLEN = 2048
KLEN = 2048
DEPTH = 64
EXT = 4096
LO_PAD = 1793
HI_START = 2302

NCORES = 2
NSUB = 16
NWORKERS = NCORES * NSUB
ROWS_PER_W = QLEN // NWORKERS
RING = 8


def _build_ext_kernel(emb_ref, ext_ref):
    ext_ref[0:LO_PAD, :] = jnp.broadcast_to(emb_ref[0:1, :], (LO_PAD, DEPTH))
    ext_ref[LO_PAD:HI_START, :] = emb_ref[:, :]
    ext_ref[HI_START:EXT, :] = jnp.broadcast_to(
        emb_ref[508:509, :], (EXT - HI_START, DEPTH))


def _sc_expand(ext_hbm, out_hbm, ext_sp, sem):
    c = lax.axis_index("c")
    s = lax.axis_index("s")

    @pl.when(s == 0)
    def _():
        pltpu.sync_copy(ext_hbm, ext_sp)

    plsc.subcore_barrier()

    base = (c * NSUB + s) * ROWS_PER_W

    def row_copy(i):
        q = base + i
        return pltpu.make_async_copy(
            ext_sp.at[pl.ds(QLEN - 1 - q, KLEN), :],
            out_hbm.at[q],
            sem)

    def fire(i, _):
        @pl.when(i >= RING)
        def _():
            row_copy(i - RING).wait()
        row_copy(i).start()
        return ()

    lax.fori_loop(0, ROWS_PER_W, fire, ())

    def drain(i, _):
        row_copy(ROWS_PER_W - RING + i).wait()
        return ()

    lax.fori_loop(0, RING, drain, ())


def kernel(inputs, embeddings):
    del inputs
    ext = pl.pallas_call(
        _build_ext_kernel,
        out_shape=jax.ShapeDtypeStruct((EXT, DEPTH), jnp.float32),
    )(embeddings)

    expand = pl.kernel(
        _sc_expand,
        out_type=jax.ShapeDtypeStruct((QLEN, KLEN, DEPTH), jnp.float32),
        mesh=plsc.VectorSubcoreMesh(
            core_axis_name="c", subcore_axis_name="s"),
        scratch_types=[
            pltpu.VMEM_SHARED((EXT, DEPTH), jnp.float32),
            pltpu.SemaphoreType.DMA,
        ],
    )
    return expand(ext)


# speedup vs baseline: 1.2480x; 1.2480x over previous
import jax
import jax.numpy as jnp
from jax.experimental import pallas as pl
from jax.experimental.pallas import tpu as pltpu

MAXSPAN = 255
QLEN = 2048
KLEN = 2048
DEPTH = 64
EXT = 4160
LO_PAD = 1793
HI_START = 2302
NPH = 16
PH_ROWS = 2048
NBUF = 8


def _build_ext_kernel(emb_ref, ext_ref):
    ext_ref[0:LO_PAD, :] = jnp.broadcast_to(emb_ref[0:1, :], (LO_PAD, DEPTH))
    ext_ref[LO_PAD:HI_START, :] = emb_ref[:, :]
    ext_ref[HI_START:EXT, :] = jnp.broadcast_to(
        emb_ref[508:509, :], (EXT - HI_START, DEPTH))


def _expand_kernel(ext_a, ext_b, out_ref, phases, stage_sem, sems):
    def stage_copy(p):
        src = (ext_a.at[pl.ds(p // 2, PH_ROWS), :] if p % 2 == 0
               else ext_b.at[pl.ds((p - 1) // 2, PH_ROWS), :])
        return pltpu.make_async_copy(src, phases.at[p], stage_sem)

    for p in range(NPH):
        stage_copy(p).start()
    for p in range(NPH):
        stage_copy(p).wait()

    def row_copy(i, b):
        s = QLEN - 1 - i
        p = jax.lax.rem(s, NPH)
        t = jax.lax.div(s, NPH)
        return pltpu.make_async_copy(
            phases.at[p].at[pl.ds(8 * t, KLEN * DEPTH // 128), :],
            out_ref.at[i],
            sems.at[b])

    def loop(g, _):
        for b in range(NBUF):
            i = g * NBUF + b

            @pl.when(g >= 1)
            def _():
                row_copy(i - NBUF, b).wait()

            row_copy(i, b).start()
        return ()

    jax.lax.fori_loop(0, QLEN // NBUF, loop, ())

    for b in range(NBUF):
        row_copy(QLEN - NBUF + b, b).wait()


def kernel(inputs, embeddings):
    del inputs
    ext = pl.pallas_call(
        _build_ext_kernel,
        out_shape=jax.ShapeDtypeStruct((EXT, DEPTH), jnp.float32),
    )(embeddings)

    ext_flat = ext.reshape(-1)
    ext_a = ext_flat.reshape(EXT * DEPTH // 128, 128)
    ext_b = ext_flat[64:64 + (EXT * DEPTH // 128 - 1) * 128].reshape(
        EXT * DEPTH // 128 - 1, 128)

    out = pl.pallas_call(
        _expand_kernel,
        in_specs=[pl.BlockSpec(memory_space=pl.ANY),
                  pl.BlockSpec(memory_space=pl.ANY)],
        out_specs=pl.BlockSpec(memory_space=pl.ANY),
        out_shape=jax.ShapeDtypeStruct(
            (QLEN, KLEN * DEPTH // 128, 128), jnp.float32),
        scratch_shapes=[
            pltpu.VMEM((NPH, PH_ROWS, 128), jnp.float32),
            pltpu.SemaphoreType.DMA,
            pltpu.SemaphoreType.DMA((NBUF,)),
        ],
    )(ext_a, ext_b)
    return out.reshape(QLEN, KLEN, DEPTH)
